# initial kernel scaffold (unmeasured)
import jax
import jax.numpy as jnp
from jax import lax
from jax.experimental import pallas as pl
from jax.experimental.pallas import tpu as pltpu

N_DEV = 8
M = 4096
N_COLS = 8192
CH = M // N_DEV


def _allreduce_body(p_ref, out_ref, comm_ref, tmp_ref, send_sems, recv_sems,
                    copy_sem, credit_sem):
    my = lax.axis_index("i")
    left = jnp.mod(my - 1, N_DEV)
    right = jnp.mod(my + 1, N_DEV)

    barrier_sem = pltpu.get_barrier_semaphore()
    for nbr in (left, right):
        pl.semaphore_signal(
            barrier_sem, inc=1,
            device_id=(nbr,), device_id_type=pl.DeviceIdType.MESH,
        )
    pl.semaphore_wait(barrier_sem, 2)

    cp = pltpu.make_async_copy(
        p_ref.at[pl.ds(my * CH, CH), :], comm_ref.at[0], copy_sem)
    cp.start()
    cp.wait()

    for t in range(2 * (N_DEV - 1)):
        s = t % 2
        r = (t + 1) % 2
        if t >= 2:
            pl.semaphore_wait(credit_sem, 1)
        rdma = pltpu.make_async_remote_copy(
            src_ref=comm_ref.at[s],
            dst_ref=comm_ref.at[r],
            send_sem=send_sems.at[s],
            recv_sem=recv_sems.at[r],
            device_id=(right,),
            device_id_type=pl.DeviceIdType.MESH,
        )
        rdma.start()
        rdma.wait()
        if 1 <= t <= 2 * (N_DEV - 1) - 2:
            pl.semaphore_signal(
                credit_sem, inc=1,
                device_id=(left,), device_id_type=pl.DeviceIdType.MESH,
            )
        if t < N_DEV - 1:
            c = jnp.mod(my - t - 1, N_DEV)
            cp = pltpu.make_async_copy(
                p_ref.at[pl.ds(c * CH, CH), :], tmp_ref, copy_sem)
            cp.start()
            cp.wait()
            comm_ref[r] = comm_ref[r] + tmp_ref[:, :]
            if t == N_DEV - 2:
                cp = pltpu.make_async_copy(
                    comm_ref.at[r], out_ref.at[pl.ds(c * CH, CH), :], copy_sem)
                cp.start()
                cp.wait()
        else:
            h = t - (N_DEV - 1)
            o = jnp.mod(my - h, N_DEV)
            cp = pltpu.make_async_copy(
                comm_ref.at[r], out_ref.at[pl.ds(o * CH, CH), :], copy_sem)
            cp.start()
            cp.wait()


def _ring_allreduce(partial):
    return pl.pallas_call(
        _allreduce_body,
        out_shape=jax.ShapeDtypeStruct((M, N_COLS), jnp.float32),
        in_specs=[pl.BlockSpec(memory_space=pltpu.ANY)],
        out_specs=pl.BlockSpec(memory_space=pltpu.ANY),
        scratch_shapes=[
            pltpu.VMEM((2, CH, N_COLS), jnp.float32),
            pltpu.VMEM((CH, N_COLS), jnp.float32),
            pltpu.SemaphoreType.DMA((2,)),
            pltpu.SemaphoreType.DMA((2,)),
            pltpu.SemaphoreType.DMA,
            pltpu.SemaphoreType.REGULAR,
        ],
        compiler_params=pltpu.CompilerParams(collective_id=0),
    )(partial)


def kernel(x, w_mat):
    partial = lax.dot_general(
        x, w_mat,
        dimension_numbers=(((1,), (0,)), ((), ())),
        precision=lax.Precision.HIGHEST,
        preferred_element_type=jnp.float32,
    )
    y = _ring_allreduce(partial)
    amax = jnp.max(jnp.abs(y))
    scale = amax / 127.0
    q = jnp.clip(jnp.round(y / scale), -127.0, 127.0)
    return (q * scale).astype(jnp.float32)


# baseline (device time: 3037294 ns/iter reference)
import jax
import jax.numpy as jnp
from jax import lax
from jax.experimental import pallas as pl
from jax.experimental.pallas import tpu as pltpu

N_DEV = 8
M = 4096
N_COLS = 8192
CH = M // N_DEV
STRIP = 2048


def _allreduce_body(p_ref, out_ref, comm_ref, tmp_ref, send_sems, recv_sems,
                    copy_sem, credit_sem):
    my = lax.axis_index("i")
    left = jnp.mod(my - 1, N_DEV)
    right = jnp.mod(my + 1, N_DEV)

    barrier_sem = pltpu.get_barrier_semaphore()
    for nbr in (left, right):
        pl.semaphore_signal(
            barrier_sem, inc=1,
            device_id=(nbr,), device_id_type=pl.DeviceIdType.MESH,
        )
    pl.semaphore_wait(barrier_sem, 2)

    cp = pltpu.make_async_copy(
        p_ref.at[pl.ds(my * CH, CH), :], comm_ref.at[0], copy_sem)
    cp.start()
    cp.wait()

    for t in range(2 * (N_DEV - 1)):
        s = t % 2
        r = (t + 1) % 2
        if t >= 2:
            pl.semaphore_wait(credit_sem, 1)
        rdma = pltpu.make_async_remote_copy(
            src_ref=comm_ref.at[s],
            dst_ref=comm_ref.at[r],
            send_sem=send_sems.at[s],
            recv_sem=recv_sems.at[r],
            device_id=(right,),
            device_id_type=pl.DeviceIdType.MESH,
        )
        rdma.start()
        rdma.wait()
        if 1 <= t <= 2 * (N_DEV - 1) - 2:
            pl.semaphore_signal(
                credit_sem, inc=1,
                device_id=(left,), device_id_type=pl.DeviceIdType.MESH,
            )
        if t < N_DEV - 1:
            c = jnp.mod(my - t - 1, N_DEV)
            for k in range(N_COLS // STRIP):
                cp = pltpu.make_async_copy(
                    p_ref.at[pl.ds(c * CH, CH), pl.ds(k * STRIP, STRIP)],
                    tmp_ref, copy_sem)
                cp.start()
                cp.wait()
                comm_ref[r, :, k * STRIP:(k + 1) * STRIP] = (
                    comm_ref[r, :, k * STRIP:(k + 1) * STRIP] + tmp_ref[:, :])
            if t == N_DEV - 2:
                cp = pltpu.make_async_copy(
                    comm_ref.at[r], out_ref.at[pl.ds(c * CH, CH), :], copy_sem)
                cp.start()
                cp.wait()
        else:
            h = t - (N_DEV - 1)
            o = jnp.mod(my - h, N_DEV)
            cp = pltpu.make_async_copy(
                comm_ref.at[r], out_ref.at[pl.ds(o * CH, CH), :], copy_sem)
            cp.start()
            cp.wait()


def _ring_allreduce(partial):
    return pl.pallas_call(
        _allreduce_body,
        out_shape=jax.ShapeDtypeStruct((M, N_COLS), jnp.float32),
        in_specs=[pl.BlockSpec(memory_space=pl.ANY)],
        out_specs=pl.BlockSpec(memory_space=pl.ANY),
        scratch_shapes=[
            pltpu.VMEM((2, CH, N_COLS), jnp.float32),
            pltpu.VMEM((CH, STRIP), jnp.float32),
            pltpu.SemaphoreType.DMA((2,)),
            pltpu.SemaphoreType.DMA((2,)),
            pltpu.SemaphoreType.DMA,
            pltpu.SemaphoreType.REGULAR,
        ],
        compiler_params=pltpu.CompilerParams(
            collective_id=0, vmem_limit_bytes=60 * 1024 * 1024),
    )(partial)


def kernel(x, w_mat):
    partial = lax.dot_general(
        x, w_mat,
        dimension_numbers=(((1,), (0,)), ((), ())),
        precision=lax.Precision.HIGHEST,
        preferred_element_type=jnp.float32,
    )
    y = _ring_allreduce(partial)
    amax = jnp.max(jnp.abs(y))
    scale = amax / 127.0
    q = jnp.clip(jnp.round(y / scale), -127.0, 127.0)
    return (q * scale).astype(jnp.float32)


# device time: 1756063 ns/iter; 1.7296x vs baseline; 1.7296x over previous
import jax
import jax.numpy as jnp
from jax import lax
from jax.experimental import pallas as pl
from jax.experimental.pallas import tpu as pltpu

N_DEV = 8
M = 4096
N_COLS = 8192
CH = M // N_DEV
HALF = CH // 2
STRIP = 2048
N_STRIPS = N_COLS // STRIP


def _allreduce_body(p_ref, out_ref, commA_ref, commB_ref, tmp_ref,
                    sendA, recvA, sendB, recvB, copy_sems,
                    creditA, creditB):
    my = lax.axis_index("i")
    left = jnp.mod(my - 1, N_DEV)
    right = jnp.mod(my + 1, N_DEV)

    barrier_sem = pltpu.get_barrier_semaphore()
    for nbr in (left, right):
        pl.semaphore_signal(
            barrier_sem, inc=1,
            device_id=(nbr,), device_id_type=pl.DeviceIdType.MESH,
        )
    pl.semaphore_wait(barrier_sem, 2)

    seedA = pltpu.make_async_copy(
        p_ref.at[pl.ds(my * CH, HALF), :], commA_ref.at[0], copy_sems.at[0])
    seedB = pltpu.make_async_copy(
        p_ref.at[pl.ds(my * CH + HALF, HALF), :], commB_ref.at[0],
        copy_sems.at[1])
    seedA.start()
    seedB.start()
    seedA.wait()
    seedB.wait()

    for t in range(2 * (N_DEV - 1)):
        s = t % 2
        r = (t + 1) % 2
        if t >= 2:
            pl.semaphore_wait(creditA, 1)
            pl.semaphore_wait(creditB, 1)
        rdmaA = pltpu.make_async_remote_copy(
            src_ref=commA_ref.at[s], dst_ref=commA_ref.at[r],
            send_sem=sendA.at[s], recv_sem=recvA.at[r],
            device_id=(right,), device_id_type=pl.DeviceIdType.MESH,
        )
        rdmaB = pltpu.make_async_remote_copy(
            src_ref=commB_ref.at[s], dst_ref=commB_ref.at[r],
            send_sem=sendB.at[s], recv_sem=recvB.at[r],
            device_id=(left,), device_id_type=pl.DeviceIdType.MESH,
        )
        rdmaA.start()
        rdmaB.start()

        is_rs = t < N_DEV - 1
        descs = [None] * (2 * N_STRIPS)
        if is_rs:
            cA = jnp.mod(my - t - 1, N_DEV)
            cB = jnp.mod(my + t + 1, N_DEV)
            rowA = cA * CH
            rowB = cB * CH + HALF
            jobs = [(rowA, k) for k in range(N_STRIPS)] + [
                (rowB, k) for k in range(N_STRIPS)]

            def start_load(j, slot):
                row, k = jobs[j]
                d = pltpu.make_async_copy(
                    p_ref.at[pl.ds(row, HALF), pl.ds(k * STRIP, STRIP)],
                    tmp_ref.at[slot], copy_sems.at[slot])
                descs[j] = d
                d.start()

            start_load(0, 0)
            start_load(1, 1)

        rdmaA.wait()
        rdmaB.wait()
        if 1 <= t <= 2 * (N_DEV - 1) - 2:
            pl.semaphore_signal(
                creditA, inc=1,
                device_id=(left,), device_id_type=pl.DeviceIdType.MESH,
            )
            pl.semaphore_signal(
                creditB, inc=1,
                device_id=(right,), device_id_type=pl.DeviceIdType.MESH,
            )

        if is_rs:
            for j in range(2 * N_STRIPS):
                slot = j % 2
                cref = commA_ref if j < N_STRIPS else commB_ref
                _, k = jobs[j]
                descs[j].wait()
                cref[r, :, k * STRIP:(k + 1) * STRIP] = (
                    cref[r, :, k * STRIP:(k + 1) * STRIP] + tmp_ref[slot])
                if j + 2 < 2 * N_STRIPS:
                    start_load(j + 2, slot)
            if t == N_DEV - 2:
                stA = pltpu.make_async_copy(
                    commA_ref.at[r], out_ref.at[pl.ds(cA * CH, HALF), :],
                    copy_sems.at[0])
                stB = pltpu.make_async_copy(
                    commB_ref.at[r], out_ref.at[pl.ds(cB * CH + HALF, HALF), :],
                    copy_sems.at[1])
                stA.start()
                stB.start()
                stA.wait()
                stB.wait()
        else:
            h = t - (N_DEV - 1)
            oA = jnp.mod(my - h, N_DEV)
            oB = jnp.mod(my + h, N_DEV)
            stA = pltpu.make_async_copy(
                commA_ref.at[r], out_ref.at[pl.ds(oA * CH, HALF), :],
                copy_sems.at[0])
            stB = pltpu.make_async_copy(
                commB_ref.at[r], out_ref.at[pl.ds(oB * CH + HALF, HALF), :],
                copy_sems.at[1])
            stA.start()
            stB.start()
            stA.wait()
            stB.wait()


def _ring_allreduce(partial):
    return pl.pallas_call(
        _allreduce_body,
        out_shape=jax.ShapeDtypeStruct((M, N_COLS), jnp.float32),
        in_specs=[pl.BlockSpec(memory_space=pl.ANY)],
        out_specs=pl.BlockSpec(memory_space=pl.ANY),
        scratch_shapes=[
            pltpu.VMEM((2, HALF, N_COLS), jnp.float32),
            pltpu.VMEM((2, HALF, N_COLS), jnp.float32),
            pltpu.VMEM((2, HALF, STRIP), jnp.float32),
            pltpu.SemaphoreType.DMA((2,)),
            pltpu.SemaphoreType.DMA((2,)),
            pltpu.SemaphoreType.DMA((2,)),
            pltpu.SemaphoreType.DMA((2,)),
            pltpu.SemaphoreType.DMA((2,)),
            pltpu.SemaphoreType.REGULAR,
            pltpu.SemaphoreType.REGULAR,
        ],
        compiler_params=pltpu.CompilerParams(
            collective_id=0, vmem_limit_bytes=60 * 1024 * 1024),
    )(partial)


def kernel(x, w_mat):
    partial = lax.dot_general(
        x, w_mat,
        dimension_numbers=(((1,), (0,)), ((), ())),
        precision=lax.Precision.HIGHEST,
        preferred_element_type=jnp.float32,
    )
    y = _ring_allreduce(partial)
    amax = jnp.max(jnp.abs(y))
    scale = amax / 127.0
    q = jnp.clip(jnp.round(y / scale), -127.0, 127.0)
    return (q * scale).astype(jnp.float32)


# device time: 1227856 ns/iter; 2.4737x vs baseline; 1.4302x over previous
import jax
import jax.numpy as jnp
from jax import lax
from jax.experimental import pallas as pl
from jax.experimental.pallas import tpu as pltpu

N_DEV = 8
M = 4096
N_COLS = 8192
CH = M // N_DEV
HALF = CH // 2
STRIP = 2048
N_STRIPS = N_COLS // STRIP


def _allreduce_body(p_ref, out_ref, commA_ref, commB_ref, qA_ref, qB_ref,
                    tmp_ref, scal_send_ref, scal_recv_ref,
                    sendA, recvA, sendB, recvB, copy_sems,
                    storeA_sems, storeB_sems, scal_send_sems, scal_recv_sems,
                    creditA, creditB):
    my = lax.axis_index("i")
    left = jnp.mod(my - 1, N_DEV)
    right = jnp.mod(my + 1, N_DEV)

    barrier_sem = pltpu.get_barrier_semaphore()
    for nbr in (left, right):
        pl.semaphore_signal(
            barrier_sem, inc=1,
            device_id=(nbr,), device_id_type=pl.DeviceIdType.MESH,
        )
    pl.semaphore_wait(barrier_sem, 2)

    seedA = pltpu.make_async_copy(
        p_ref.at[pl.ds(my * CH, HALF), :], commA_ref.at[0], copy_sems.at[0])
    seedB = pltpu.make_async_copy(
        p_ref.at[pl.ds(my * CH + HALF, HALF), :], commB_ref.at[0],
        copy_sems.at[1])
    seedA.start()
    seedB.start()
    seedA.wait()
    seedB.wait()

    scale = None
    store_descs = {}

    for t in range(2 * (N_DEV - 1)):
        s = t % 2
        r = (t + 1) % 2
        is_rs = t < N_DEV - 1
        if t >= 2:
            pl.semaphore_wait(creditA, 1)
            pl.semaphore_wait(creditB, 1)
        srcA, srcB = (commA_ref, commB_ref) if is_rs else (qA_ref, qB_ref)
        rdmaA = pltpu.make_async_remote_copy(
            src_ref=srcA.at[s], dst_ref=srcA.at[r],
            send_sem=sendA.at[s], recv_sem=recvA.at[r],
            device_id=(right,), device_id_type=pl.DeviceIdType.MESH,
        )
        rdmaB = pltpu.make_async_remote_copy(
            src_ref=srcB.at[s], dst_ref=srcB.at[r],
            send_sem=sendB.at[s], recv_sem=recvB.at[r],
            device_id=(left,), device_id_type=pl.DeviceIdType.MESH,
        )
        rdmaA.start()
        rdmaB.start()

        descs = [None] * (2 * N_STRIPS)
        if is_rs:
            cA = jnp.mod(my - t - 1, N_DEV)
            cB = jnp.mod(my + t + 1, N_DEV)
            jobs = [(cA * CH, k) for k in range(N_STRIPS)] + [
                (cB * CH + HALF, k) for k in range(N_STRIPS)]

            def start_load(j, slot):
                row, k = jobs[j]
                d = pltpu.make_async_copy(
                    p_ref.at[pl.ds(row, HALF), pl.ds(k * STRIP, STRIP)],
                    tmp_ref.at[slot], copy_sems.at[slot])
                descs[j] = d
                d.start()

            start_load(0, 0)
            start_load(1, 1)
        else:
            h = t - (N_DEV - 1)
            if h == 0:
                oA = jnp.mod(my + 1, N_DEV)
                oB = jnp.mod(my - 1, N_DEV)
            else:
                oA = jnp.mod(my - (h - 1), N_DEV)
                oB = jnp.mod(my + (h - 1), N_DEV)
            for ring, qref, cref, row, ssems in (
                    ("A", qA_ref, commA_ref, oA * CH, storeA_sems),
                    ("B", qB_ref, commB_ref, oB * CH + HALF, storeB_sems)):
                prev = store_descs.pop((ring, s), None)
                if prev is not None:
                    prev.wait()
                cref[s] = qref[s].astype(jnp.float32) * scale
                st = pltpu.make_async_copy(
                    cref.at[s], out_ref.at[pl.ds(row, HALF), :], ssems.at[s])
                st.start()
                store_descs[(ring, s)] = st

        rdmaA.wait()
        rdmaB.wait()
        if 1 <= t <= 2 * (N_DEV - 1) - 2:
            pl.semaphore_signal(
                creditA, inc=1,
                device_id=(left,), device_id_type=pl.DeviceIdType.MESH,
            )
            pl.semaphore_signal(
                creditB, inc=1,
                device_id=(right,), device_id_type=pl.DeviceIdType.MESH,
            )

        if is_rs:
            for j in range(2 * N_STRIPS):
                slot = j % 2
                cref = commA_ref if j < N_STRIPS else commB_ref
                _, k = jobs[j]
                descs[j].wait()
                cref[r, :, k * STRIP:(k + 1) * STRIP] = (
                    cref[r, :, k * STRIP:(k + 1) * STRIP] + tmp_ref[slot])
                if j + 2 < 2 * N_STRIPS:
                    start_load(j + 2, slot)

            if t == N_DEV - 2:
                local_amax = jnp.maximum(
                    jnp.max(jnp.abs(commA_ref[r])),
                    jnp.max(jnp.abs(commB_ref[r])))
                acc = local_amax
                for h in range(N_DEV - 1):
                    scal_send_ref[...] = jnp.full((1, 128), acc, jnp.float32)
                    sd = pltpu.make_async_remote_copy(
                        src_ref=scal_send_ref,
                        dst_ref=scal_recv_ref.at[h],
                        send_sem=scal_send_sems.at[h],
                        recv_sem=scal_recv_sems.at[h],
                        device_id=(right,),
                        device_id_type=pl.DeviceIdType.MESH,
                    )
                    sd.start()
                    sd.wait()
                    acc = jnp.maximum(acc, scal_recv_ref[h, 0, 0])
                scale = acc / 127.0
                qA_ref[r] = jnp.clip(
                    jnp.round(commA_ref[r] / scale), -127.0, 127.0
                ).astype(jnp.int8)
                qB_ref[r] = jnp.clip(
                    jnp.round(commB_ref[r] / scale), -127.0, 127.0
                ).astype(jnp.int8)

    for ring, qref, cref, row, ssems in (
            ("A", qA_ref, commA_ref, jnp.mod(my - 6, N_DEV) * CH,
             storeA_sems),
            ("B", qB_ref, commB_ref,
             jnp.mod(my + 6, N_DEV) * CH + HALF, storeB_sems)):
        prev = store_descs.pop((ring, 0), None)
        if prev is not None:
            prev.wait()
        cref[0] = qref[0].astype(jnp.float32) * scale
        st = pltpu.make_async_copy(
            cref.at[0], out_ref.at[pl.ds(row, HALF), :], ssems.at[0])
        st.start()
        store_descs[(ring, 0)] = st
    for d in store_descs.values():
        d.wait()


def _fused_allreduce_quant(partial):
    return pl.pallas_call(
        _allreduce_body,
        out_shape=jax.ShapeDtypeStruct((M, N_COLS), jnp.float32),
        in_specs=[pl.BlockSpec(memory_space=pl.ANY)],
        out_specs=pl.BlockSpec(memory_space=pl.ANY),
        scratch_shapes=[
            pltpu.VMEM((2, HALF, N_COLS), jnp.float32),
            pltpu.VMEM((2, HALF, N_COLS), jnp.float32),
            pltpu.VMEM((2, HALF, N_COLS), jnp.int8),
            pltpu.VMEM((2, HALF, N_COLS), jnp.int8),
            pltpu.VMEM((2, HALF, STRIP), jnp.float32),
            pltpu.VMEM((1, 128), jnp.float32),
            pltpu.VMEM((N_DEV - 1, 1, 128), jnp.float32),
            pltpu.SemaphoreType.DMA((2,)),
            pltpu.SemaphoreType.DMA((2,)),
            pltpu.SemaphoreType.DMA((2,)),
            pltpu.SemaphoreType.DMA((2,)),
            pltpu.SemaphoreType.DMA((2,)),
            pltpu.SemaphoreType.DMA((2,)),
            pltpu.SemaphoreType.DMA((2,)),
            pltpu.SemaphoreType.DMA((N_DEV - 1,)),
            pltpu.SemaphoreType.DMA((N_DEV - 1,)),
            pltpu.SemaphoreType.REGULAR,
            pltpu.SemaphoreType.REGULAR,
        ],
        compiler_params=pltpu.CompilerParams(
            collective_id=0, vmem_limit_bytes=62 * 1024 * 1024),
    )(partial)


def kernel(x, w_mat):
    partial = lax.dot_general(
        x, w_mat,
        dimension_numbers=(((1,), (0,)), ((), ())),
        precision=lax.Precision.HIGHEST,
        preferred_element_type=jnp.float32,
    )
    return _fused_allreduce_quant(partial)


# device time: 914605 ns/iter; 3.3209x vs baseline; 1.3425x over previous
import jax
import jax.numpy as jnp
from jax import lax
from jax.experimental import pallas as pl
from jax.experimental.pallas import tpu as pltpu

N_DEV = 8
M = 4096
N_COLS = 8192
CH = M // N_DEV
HALF = CH // 2
STRIP = 2048
N_STRIPS = N_COLS // STRIP


def _allreduce_body(p_ref, out_ref, commA_ref, commB_ref, qA_ref, qB_ref,
                    tmp_ref, stage_ref, scal_send_ref, scal_recv_ref,
                    sendA, recvA, sendB, recvB, copy_sems,
                    storeA_sems, storeB_sems, scal_send_sems, scal_recv_sems,
                    creditA, creditB):
    my = lax.axis_index("i")
    left = jnp.mod(my - 1, N_DEV)
    right = jnp.mod(my + 1, N_DEV)

    barrier_sem = pltpu.get_barrier_semaphore()
    for nbr in (left, right):
        pl.semaphore_signal(
            barrier_sem, inc=1,
            device_id=(nbr,), device_id_type=pl.DeviceIdType.MESH,
        )
    pl.semaphore_wait(barrier_sem, 2)

    seed_jobs = [(my * CH, k, commA_ref) for k in range(N_STRIPS)] + [
        (my * CH + HALF, k, commB_ref) for k in range(N_STRIPS)]
    seed_descs = [None] * len(seed_jobs)

    def seed_load(j, slot):
        row, k, _ = seed_jobs[j]
        d = pltpu.make_async_copy(
            p_ref.at[pl.ds(row, HALF), pl.ds(k * STRIP, STRIP)],
            tmp_ref.at[slot], copy_sems.at[slot])
        seed_descs[j] = d
        d.start()

    seed_load(0, 0)
    seed_load(1, 1)
    for j in range(len(seed_jobs)):
        slot = j % 2
        _, k, cref = seed_jobs[j]
        seed_descs[j].wait()
        cref[0, :, k * STRIP:(k + 1) * STRIP] = tmp_ref[slot].astype(
            jnp.bfloat16)
        if j + 2 < len(seed_jobs):
            seed_load(j + 2, slot)

    scale = None
    store_descs = {}

    for t in range(2 * (N_DEV - 1)):
        s = t % 2
        r = (t + 1) % 2
        is_rs = t < N_DEV - 1
        if t >= 2:
            pl.semaphore_wait(creditA, 1)
            pl.semaphore_wait(creditB, 1)
        srcA, srcB = (commA_ref, commB_ref) if is_rs else (qA_ref, qB_ref)
        rdmaA = pltpu.make_async_remote_copy(
            src_ref=srcA.at[s], dst_ref=srcA.at[r],
            send_sem=sendA.at[s], recv_sem=recvA.at[r],
            device_id=(right,), device_id_type=pl.DeviceIdType.MESH,
        )
        rdmaB = pltpu.make_async_remote_copy(
            src_ref=srcB.at[s], dst_ref=srcB.at[r],
            send_sem=sendB.at[s], recv_sem=recvB.at[r],
            device_id=(left,), device_id_type=pl.DeviceIdType.MESH,
        )
        rdmaA.start()
        rdmaB.start()

        descs = [None] * (2 * N_STRIPS)
        if is_rs:
            cA = jnp.mod(my - t - 1, N_DEV)
            cB = jnp.mod(my + t + 1, N_DEV)
            jobs = [(cA * CH, k) for k in range(N_STRIPS)] + [
                (cB * CH + HALF, k) for k in range(N_STRIPS)]

            def start_load(j, slot):
                row, k = jobs[j]
                d = pltpu.make_async_copy(
                    p_ref.at[pl.ds(row, HALF), pl.ds(k * STRIP, STRIP)],
                    tmp_ref.at[slot], copy_sems.at[slot])
                descs[j] = d
                d.start()

            start_load(0, 0)
            start_load(1, 1)
        else:
            h = t - (N_DEV - 1)
            if h == 0:
                oA = jnp.mod(my + 1, N_DEV)
                oB = jnp.mod(my - 1, N_DEV)
            else:
                oA = jnp.mod(my - (h - 1), N_DEV)
                oB = jnp.mod(my + (h - 1), N_DEV)
            for ring, stslot, qref, row, ssems in (
                    ("A", 0, qA_ref, oA * CH, storeA_sems),
                    ("B", 1, qB_ref, oB * CH + HALF, storeB_sems)):
                prev = store_descs.pop(ring, None)
                if prev is not None:
                    prev.wait()
                stage_ref[stslot] = qref[s].astype(jnp.float32) * scale
                st = pltpu.make_async_copy(
                    stage_ref.at[stslot], out_ref.at[pl.ds(row, HALF), :],
                    ssems.at[stslot])
                st.start()
                store_descs[ring] = st

        rdmaA.wait()
        rdmaB.wait()
        if 1 <= t <= 2 * (N_DEV - 1) - 2:
            pl.semaphore_signal(
                creditA, inc=1,
                device_id=(left,), device_id_type=pl.DeviceIdType.MESH,
            )
            pl.semaphore_signal(
                creditB, inc=1,
                device_id=(right,), device_id_type=pl.DeviceIdType.MESH,
            )

        if is_rs:
            for j in range(2 * N_STRIPS):
                slot = j % 2
                cref = commA_ref if j < N_STRIPS else commB_ref
                _, k = jobs[j]
                descs[j].wait()
                cref[r, :, k * STRIP:(k + 1) * STRIP] = (
                    cref[r, :, k * STRIP:(k + 1) * STRIP].astype(jnp.float32)
                    + tmp_ref[slot]).astype(jnp.bfloat16)
                if j + 2 < 2 * N_STRIPS:
                    start_load(j + 2, slot)

            if t == N_DEV - 2:
                local_amax = jnp.maximum(
                    jnp.max(jnp.abs(commA_ref[r]).astype(jnp.float32)),
                    jnp.max(jnp.abs(commB_ref[r]).astype(jnp.float32)))
                acc = local_amax
                for h in range(N_DEV - 1):
                    scal_send_ref[...] = jnp.full((1, 128), acc, jnp.float32)
                    sd = pltpu.make_async_remote_copy(
                        src_ref=scal_send_ref,
                        dst_ref=scal_recv_ref.at[h],
                        send_sem=scal_send_sems.at[h],
                        recv_sem=scal_recv_sems.at[h],
                        device_id=(right,),
                        device_id_type=pl.DeviceIdType.MESH,
                    )
                    sd.start()
                    sd.wait()
                    acc = jnp.maximum(acc, scal_recv_ref[h, 0, 0])
                scale = acc / 127.0
                qA_ref[r] = jnp.clip(
                    jnp.round(commA_ref[r].astype(jnp.float32) / scale),
                    -127.0, 127.0).astype(jnp.int8)
                qB_ref[r] = jnp.clip(
                    jnp.round(commB_ref[r].astype(jnp.float32) / scale),
                    -127.0, 127.0).astype(jnp.int8)

    for ring, stslot, qref, row, ssems in (
            ("A", 0, qA_ref, jnp.mod(my - 6, N_DEV) * CH, storeA_sems),
            ("B", 1, qB_ref,
             jnp.mod(my + 6, N_DEV) * CH + HALF, storeB_sems)):
        prev = store_descs.pop(ring, None)
        if prev is not None:
            prev.wait()
        stage_ref[stslot] = qref[0].astype(jnp.float32) * scale
        st = pltpu.make_async_copy(
            stage_ref.at[stslot], out_ref.at[pl.ds(row, HALF), :],
            ssems.at[stslot])
        st.start()
        store_descs[ring] = st
    for d in store_descs.values():
        d.wait()


def _fused_allreduce_quant(partial):
    return pl.pallas_call(
        _allreduce_body,
        out_shape=jax.ShapeDtypeStruct((M, N_COLS), jnp.float32),
        in_specs=[pl.BlockSpec(memory_space=pl.ANY)],
        out_specs=pl.BlockSpec(memory_space=pl.ANY),
        scratch_shapes=[
            pltpu.VMEM((2, HALF, N_COLS), jnp.bfloat16),
            pltpu.VMEM((2, HALF, N_COLS), jnp.bfloat16),
            pltpu.VMEM((2, HALF, N_COLS), jnp.int8),
            pltpu.VMEM((2, HALF, N_COLS), jnp.int8),
            pltpu.VMEM((2, HALF, STRIP), jnp.float32),
            pltpu.VMEM((2, HALF, N_COLS), jnp.float32),
            pltpu.VMEM((1, 128), jnp.float32),
            pltpu.VMEM((N_DEV - 1, 1, 128), jnp.float32),
            pltpu.SemaphoreType.DMA((2,)),
            pltpu.SemaphoreType.DMA((2,)),
            pltpu.SemaphoreType.DMA((2,)),
            pltpu.SemaphoreType.DMA((2,)),
            pltpu.SemaphoreType.DMA((2,)),
            pltpu.SemaphoreType.DMA((2,)),
            pltpu.SemaphoreType.DMA((2,)),
            pltpu.SemaphoreType.DMA((N_DEV - 1,)),
            pltpu.SemaphoreType.DMA((N_DEV - 1,)),
            pltpu.SemaphoreType.REGULAR,
            pltpu.SemaphoreType.REGULAR,
        ],
        compiler_params=pltpu.CompilerParams(
            collective_id=0, vmem_limit_bytes=62 * 1024 * 1024),
    )(partial)


def kernel(x, w_mat):
    partial = lax.dot_general(
        x, w_mat,
        dimension_numbers=(((1,), (0,)), ((), ())),
        precision=lax.Precision.HIGHEST,
        preferred_element_type=jnp.float32,
    )
    return _fused_allreduce_quant(partial)


# device time: 674164 ns/iter; 4.5053x vs baseline; 1.3567x over previous
import jax
import jax.numpy as jnp
from jax import lax
from jax.experimental import pallas as pl
from jax.experimental.pallas import tpu as pltpu

N_DEV = 8
M = 4096
K = 512
N_COLS = 8192
CH = M // N_DEV
HALF = CH // 2
STRIP = 2048
N_STRIPS = N_COLS // STRIP


def _body(x_ref, w_ref, out_ref, commA_ref, commB_ref, qA_ref, qB_ref,
          w_vmem, xA_buf, xB_buf, stage_ref, scal_send_ref, scal_recv_ref,
          sendA, recvA, sendB, recvB, copy_sems,
          storeA_sems, storeB_sems, scal_send_sems, scal_recv_sems,
          creditA, creditB):
    my = lax.axis_index("i")
    left = jnp.mod(my - 1, N_DEV)
    right = jnp.mod(my + 1, N_DEV)

    barrier_sem = pltpu.get_barrier_semaphore()
    for nbr in (left, right):
        pl.semaphore_signal(
            barrier_sem, inc=1,
            device_id=(nbr,), device_id_type=pl.DeviceIdType.MESH,
        )
    pl.semaphore_wait(barrier_sem, 2)

    wld0 = pltpu.make_async_copy(
        w_ref.at[pl.ds(0, HALF), :], stage_ref.at[0], copy_sems.at[0])
    wld1 = pltpu.make_async_copy(
        w_ref.at[pl.ds(HALF, HALF), :], stage_ref.at[1], copy_sems.at[1])
    wld0.start()
    wld1.start()

    def load_x(row, buf, sem_idx):
        d = pltpu.make_async_copy(
            x_ref.at[pl.ds(row, HALF), :], buf, copy_sems.at[sem_idx])
        d.start()
        return d

    wld0.wait()
    w_vmem[0:HALF, :] = stage_ref[0].astype(jnp.bfloat16)
    wld1.wait()
    w_vmem[HALF:K, :] = stage_ref[1].astype(jnp.bfloat16)
    ldA = load_x(my * CH, xA_buf, 0)
    ldA.wait()
    ldB = load_x(my * CH + HALF, xB_buf, 1)

    def matmul(xbuf, k):
        return lax.dot_general(
            xbuf[...].astype(jnp.bfloat16),
            w_vmem[:, k * STRIP:(k + 1) * STRIP],
            dimension_numbers=(((1,), (0,)), ((), ())),
            preferred_element_type=jnp.float32,
        )

    for k in range(N_STRIPS):
        commA_ref[0, :, k * STRIP:(k + 1) * STRIP] = matmul(
            xA_buf, k).astype(jnp.bfloat16)
    ldB.wait()
    for k in range(N_STRIPS):
        commB_ref[0, :, k * STRIP:(k + 1) * STRIP] = matmul(
            xB_buf, k).astype(jnp.bfloat16)

    scale = None
    store_descs = {}

    def stage_store(qref, s, ring, origin_row, ssems):
        prev = store_descs.pop(ring, None)
        if prev is not None:
            prev.wait()
        stage_ref[ring] = qref[s].astype(jnp.float32) * scale
        st = pltpu.make_async_copy(
            stage_ref.at[ring], out_ref.at[pl.ds(origin_row, HALF), :],
            ssems.at[0])
        st.start()
        store_descs[ring] = st

    for t in range(2 * (N_DEV - 1)):
        s = t % 2
        r = (t + 1) % 2
        is_rs = t < N_DEV - 1
        if t >= 2:
            pl.semaphore_wait(creditA, 1)
            pl.semaphore_wait(creditB, 1)
        srcA, srcB = (commA_ref, commB_ref) if is_rs else (qA_ref, qB_ref)
        rdmaA = pltpu.make_async_remote_copy(
            src_ref=srcA.at[s], dst_ref=srcA.at[r],
            send_sem=sendA.at[s], recv_sem=recvA.at[r],
            device_id=(right,), device_id_type=pl.DeviceIdType.MESH,
        )
        rdmaB = pltpu.make_async_remote_copy(
            src_ref=srcB.at[s], dst_ref=srcB.at[r],
            send_sem=sendB.at[s], recv_sem=recvB.at[r],
            device_id=(left,), device_id_type=pl.DeviceIdType.MESH,
        )
        rdmaA.start()
        rdmaB.start()

        if is_rs:
            cA = jnp.mod(my - t - 1, N_DEV)
            cB = jnp.mod(my + t + 1, N_DEV)
            ldA = load_x(cA * CH, xA_buf, 0)
            ldB = load_x(cB * CH + HALF, xB_buf, 1)
        else:
            h = t - (N_DEV - 1)
            if h == 0:
                oA = jnp.mod(my + 1, N_DEV)
                oB = jnp.mod(my - 1, N_DEV)
            else:
                oA = jnp.mod(my - (h - 1), N_DEV)
                oB = jnp.mod(my + (h - 1), N_DEV)
            stage_store(qA_ref, s, 0, oA * CH, storeA_sems)
            stage_store(qB_ref, s, 1, oB * CH + HALF, storeB_sems)

        rdmaA.wait()
        rdmaB.wait()
        if 1 <= t <= 2 * (N_DEV - 1) - 2:
            pl.semaphore_signal(
                creditA, inc=1,
                device_id=(left,), device_id_type=pl.DeviceIdType.MESH,
            )
            pl.semaphore_signal(
                creditB, inc=1,
                device_id=(right,), device_id_type=pl.DeviceIdType.MESH,
            )

        if is_rs:
            ldA.wait()
            for k in range(N_STRIPS):
                commA_ref[r, :, k * STRIP:(k + 1) * STRIP] = (
                    commA_ref[r, :, k * STRIP:(k + 1) * STRIP].astype(
                        jnp.float32) + matmul(xA_buf, k)
                ).astype(jnp.bfloat16)
            ldB.wait()
            for k in range(N_STRIPS):
                commB_ref[r, :, k * STRIP:(k + 1) * STRIP] = (
                    commB_ref[r, :, k * STRIP:(k + 1) * STRIP].astype(
                        jnp.float32) + matmul(xB_buf, k)
                ).astype(jnp.bfloat16)

            if t == N_DEV - 2:
                local_amax = jnp.maximum(
                    jnp.max(jnp.abs(commA_ref[r]).astype(jnp.float32)),
                    jnp.max(jnp.abs(commB_ref[r]).astype(jnp.float32)))
                acc = local_amax
                for h in range(N_DEV - 1):
                    scal_send_ref[...] = jnp.full((1, 128), acc, jnp.float32)
                    sd = pltpu.make_async_remote_copy(
                        src_ref=scal_send_ref,
                        dst_ref=scal_recv_ref.at[h],
                        send_sem=scal_send_sems.at[h],
                        recv_sem=scal_recv_sems.at[h],
                        device_id=(right,),
                        device_id_type=pl.DeviceIdType.MESH,
                    )
                    sd.start()
                    sd.wait()
                    acc = jnp.maximum(acc, scal_recv_ref[h, 0, 0])
                scale = acc / 127.0
                qA_ref[r] = jnp.clip(
                    jnp.round(commA_ref[r].astype(jnp.float32) / scale),
                    -127.0, 127.0).astype(jnp.int8)
                qB_ref[r] = jnp.clip(
                    jnp.round(commB_ref[r].astype(jnp.float32) / scale),
                    -127.0, 127.0).astype(jnp.int8)

    stage_store(qA_ref, 0, 0, jnp.mod(my - 6, N_DEV) * CH, storeA_sems)
    stage_store(qB_ref, 0, 1, jnp.mod(my + 6, N_DEV) * CH + HALF,
                storeB_sems)
    for d in store_descs.values():
        d.wait()


def kernel(x, w_mat):
    return pl.pallas_call(
        _body,
        out_shape=jax.ShapeDtypeStruct((M, N_COLS), jnp.float32),
        in_specs=[pl.BlockSpec(memory_space=pl.ANY),
                  pl.BlockSpec(memory_space=pl.ANY)],
        out_specs=pl.BlockSpec(memory_space=pl.ANY),
        scratch_shapes=[
            pltpu.VMEM((2, HALF, N_COLS), jnp.bfloat16),
            pltpu.VMEM((2, HALF, N_COLS), jnp.bfloat16),
            pltpu.VMEM((2, HALF, N_COLS), jnp.int8),
            pltpu.VMEM((2, HALF, N_COLS), jnp.int8),
            pltpu.VMEM((K, N_COLS), jnp.bfloat16),
            pltpu.VMEM((HALF, K), jnp.float32),
            pltpu.VMEM((HALF, K), jnp.float32),
            pltpu.VMEM((2, HALF, N_COLS), jnp.float32),
            pltpu.VMEM((1, 128), jnp.float32),
            pltpu.VMEM((N_DEV - 1, 1, 128), jnp.float32),
            pltpu.SemaphoreType.DMA((2,)),
            pltpu.SemaphoreType.DMA((2,)),
            pltpu.SemaphoreType.DMA((2,)),
            pltpu.SemaphoreType.DMA((2,)),
            pltpu.SemaphoreType.DMA((2,)),
            pltpu.SemaphoreType.DMA((2,)),
            pltpu.SemaphoreType.DMA((2,)),
            pltpu.SemaphoreType.DMA((N_DEV - 1,)),
            pltpu.SemaphoreType.DMA((N_DEV - 1,)),
            pltpu.SemaphoreType.REGULAR,
            pltpu.SemaphoreType.REGULAR,
        ],
        compiler_params=pltpu.CompilerParams(
            collective_id=0, vmem_limit_bytes=62 * 1024 * 1024),
    )(x, w_mat)
